# baseline (speedup 1.0000x reference)
import functools
import jax
import jax.numpy as jnp
from jax import lax
from jax.experimental import pallas as pl
from jax.experimental.pallas import tpu as pltpu
from jax.experimental.pallas import tpu_sc as plsc

_B, _S, _D = 4, 4096, 2048
_NC, _NS, _L = 2, 16, 16
_NW = _NC * _NS
_ROWS_PER_W = _S // _NW
_CHUNK = 8
_N_CHUNKS = _ROWS_PER_W // _CHUNK


def _sc_body(x_hbm, pos_hbm, out_hbm, pos_v, x_v):
    wid = lax.axis_index("s") * _NC + lax.axis_index("c")
    base_row = wid * _ROWS_PER_W

    def chunk_body(ci, carry):
        row = base_row + ci * _CHUNK
        pltpu.sync_copy(pos_hbm.at[pl.ds(row, _CHUNK)], pos_v)
        for b in range(_B):
            pltpu.sync_copy(x_hbm.at[b, pl.ds(row, _CHUNK)], x_v)
            for r in range(_CHUNK):
                @plsc.parallel_loop(0, _D, _L, unroll=4)
                def _(i):
                    sl = pl.ds(i, _L)
                    x_v[r, sl] = x_v[r, sl] + pos_v[r, sl]
            pltpu.sync_copy(x_v, out_hbm.at[b, pl.ds(row, _CHUNK)])
        return carry

    lax.fori_loop(0, _N_CHUNKS, chunk_body, 0)


def kernel(x, pos_table):
    mesh = plsc.VectorSubcoreMesh(
        core_axis_name="c", subcore_axis_name="s",
        num_cores=_NC, num_subcores=_NS,
    )
    sc_call = pl.kernel(
        _sc_body,
        out_type=jax.ShapeDtypeStruct((_B, _S, _D), jnp.float32),
        mesh=mesh,
        scratch_types=[
            pltpu.VMEM((_CHUNK, _D), jnp.float32),
            pltpu.VMEM((_CHUNK, _D), jnp.float32),
        ],
    )
    return sc_call(x, pos_table)


# speedup vs baseline: 1.2217x; 1.2217x over previous
import functools
import jax
import jax.numpy as jnp
from jax import lax
from jax.experimental import pallas as pl
from jax.experimental.pallas import tpu as pltpu
from jax.experimental.pallas import tpu_sc as plsc

_B, _S, _D = 4, 4096, 2048
_BS = 512

_NC, _NS, _L = 2, 16, 16
_NW = _NC * _NS
_ROWS_PER_W = _S // _NW
_CHUNK = 8
_N_CHUNKS = _ROWS_PER_W // _CHUNK
_SC_BATCH = 3


def _tc_body(x_ref, pos_ref, out_ref):
    out_ref[0] = x_ref[0] + pos_ref[...]


def _tc_call(x, pos_table, n_b):
    n_s = _S // _BS
    return pl.pallas_call(
        _tc_body,
        grid=(n_s, n_b),
        in_specs=[
            pl.BlockSpec((1, _BS, _D), lambda s, b: (b, s, 0)),
            pl.BlockSpec((_BS, _D), lambda s, b: (s, 0)),
        ],
        out_specs=pl.BlockSpec((1, _BS, _D), lambda s, b: (b, s, 0)),
        out_shape=jax.ShapeDtypeStruct((n_b, _S, _D), x.dtype),
        compiler_params=pltpu.CompilerParams(
            dimension_semantics=("arbitrary", "arbitrary"),
        ),
    )(x, pos_table)


def _sc_body(x_hbm, pos_hbm, out_hbm, pos_v, x_v):
    wid = lax.axis_index("s") * _NC + lax.axis_index("c")
    base_row = wid * _ROWS_PER_W

    def chunk_body(ci, carry):
        row = base_row + ci * _CHUNK
        pltpu.sync_copy(pos_hbm.at[pl.ds(row, _CHUNK)], pos_v)
        pltpu.sync_copy(x_hbm.at[_SC_BATCH, pl.ds(row, _CHUNK)], x_v)
        for r in range(_CHUNK):
            @plsc.parallel_loop(0, _D, _L, unroll=4)
            def _(i):
                sl = pl.ds(i, _L)
                x_v[r, sl] = x_v[r, sl] + pos_v[r, sl]
        pltpu.sync_copy(x_v, out_hbm.at[pl.ds(row, _CHUNK)])
        return carry

    lax.fori_loop(0, _N_CHUNKS, chunk_body, 0)


def _sc_call(x, pos_table):
    mesh = plsc.VectorSubcoreMesh(
        core_axis_name="c", subcore_axis_name="s",
        num_cores=_NC, num_subcores=_NS,
    )
    return pl.kernel(
        _sc_body,
        out_type=jax.ShapeDtypeStruct((_S, _D), jnp.float32),
        mesh=mesh,
        scratch_types=[
            pltpu.VMEM((_CHUNK, _D), jnp.float32),
            pltpu.VMEM((_CHUNK, _D), jnp.float32),
        ],
    )(x, pos_table)


def kernel(x, pos_table):
    sc_out = _sc_call(x, pos_table)
    tc_out = _tc_call(x, pos_table, _B - 1)
    return jnp.concatenate([tc_out, sc_out[None]], axis=0)


# speedup vs baseline: 2.4587x; 2.0126x over previous
import jax
import jax.numpy as jnp
from jax.experimental import pallas as pl
from jax.experimental.pallas import tpu as pltpu

_BS = 256


def _body(x_ref, pos_ref, out_ref):
    out_ref[0] = x_ref[0] + pos_ref[...]


def kernel(x, pos_table):
    B, S, D = x.shape
    n_s = S // _BS
    return pl.pallas_call(
        _body,
        grid=(n_s, B),
        in_specs=[
            pl.BlockSpec((1, _BS, D), lambda s, b: (b, s, 0)),
            pl.BlockSpec((_BS, D), lambda s, b: (s, 0)),
        ],
        out_specs=pl.BlockSpec((1, _BS, D), lambda s, b: (b, s, 0)),
        out_shape=jax.ShapeDtypeStruct((B, S, D), x.dtype),
        compiler_params=pltpu.CompilerParams(
            dimension_semantics=("arbitrary", "arbitrary"),
        ),
    )(x, pos_table)


# speedup vs baseline: 2.8555x; 1.1614x over previous
import jax
import jax.numpy as jnp
from jax.experimental import pallas as pl
from jax.experimental.pallas import tpu as pltpu

_BS = 1024


def _body(x_ref, pos_ref, out_ref):
    out_ref[0] = x_ref[0] + pos_ref[...]


def kernel(x, pos_table):
    B, S, D = x.shape
    n_s = S // _BS
    return pl.pallas_call(
        _body,
        grid=(n_s, B),
        in_specs=[
            pl.BlockSpec((1, _BS, D), lambda s, b: (b, s, 0)),
            pl.BlockSpec((_BS, D), lambda s, b: (s, 0)),
        ],
        out_specs=pl.BlockSpec((1, _BS, D), lambda s, b: (b, s, 0)),
        out_shape=jax.ShapeDtypeStruct((B, S, D), x.dtype),
        compiler_params=pltpu.CompilerParams(
            dimension_semantics=("arbitrary", "arbitrary"),
        ),
    )(x, pos_table)
